# baseline (speedup 1.0000x reference)
import functools
import math

import jax
import jax.numpy as jnp
from jax import lax
from jax.experimental import pallas as pl
from jax.experimental.pallas import tpu as pltpu
from jax.experimental.pallas import tpu_sc as plsc

_LANES = 16


def _make_sc_gather(n_nodes, e, b_per_w, nc):
    groups = b_per_w // _LANES

    mesh = plsc.VectorSubcoreMesh(core_axis_name="c", subcore_axis_name="s")

    @functools.partial(
        pl.kernel,
        out_type=jax.ShapeDtypeStruct((8, e), jnp.float32),
        mesh=mesh,
        scratch_types=[
            pltpu.VMEM((n_nodes * 3,), jnp.float32),
            pltpu.VMEM((b_per_w,), jnp.int32),
            pltpu.VMEM((b_per_w,), jnp.int32),
            pltpu.VMEM((8, b_per_w), jnp.float32),
        ],
        compiler_params=pltpu.CompilerParams(needs_layout_passes=False),
    )
    def sc_gather(pos_hbm, i_hbm, j_hbm, out_hbm, pos_v, i_v, j_v, vec_v):
        wid = lax.axis_index("s") * nc + lax.axis_index("c")
        base = jnp.minimum(wid * b_per_w, e - b_per_w)

        pltpu.sync_copy(pos_hbm, pos_v)
        pltpu.sync_copy(i_hbm.at[pl.ds(base, b_per_w)], i_v)
        pltpu.sync_copy(j_hbm.at[pl.ds(base, b_per_w)], j_v)

        @plsc.parallel_loop(0, groups, unroll=4)
        def body(g):
            off = g * _LANES
            i_idx = i_v[pl.ds(off, _LANES)] * 3
            j_idx = j_v[pl.ds(off, _LANES)] * 3
            comps = []
            for k in range(3):
                ip = plsc.load_gather(pos_v, [i_idx + k])
                jp = plsc.load_gather(pos_v, [j_idx + k])
                comps.append(ip - jp)
            v0, v1, v2 = comps
            d2 = v0 * v0 + v1 * v1 + v2 * v2
            for k, val in enumerate((v0, v1, v2, d2)):
                vec_v[k, pl.ds(off, _LANES)] = val

        pltpu.sync_copy(vec_v, out_hbm.at[:, pl.ds(base, b_per_w)])

    return sc_gather


_C_SH = math.sqrt(3.0 / (4.0 * math.pi))
_INV_SQRT2 = 1.0 / math.sqrt(2.0)


def _tc_body(vec_ref, xj_ref, w1_ref, b1_ref, w2_ref, b2_ref, out_ref):
    v = jnp.transpose(vec_ref[...])
    vx = v[:, 0:1]
    vy = v[:, 1:2]
    vz = v[:, 2:3]
    d2 = v[:, 3:4]
    dist = jnp.sqrt(d2 + 1e-12)
    inv = (_C_SH * _INV_SQRT2) / dist
    s0 = vy * inv
    s1 = vz * inv
    s2 = vx * inv
    a = dist * w1_ref[...] + b1_ref[...]
    h = a * (1.0 / (1.0 + jnp.exp(-a)))
    r = jnp.dot(h, w2_ref[...], preferred_element_type=jnp.float32) + b2_ref[...]
    xj0 = xj_ref[0]
    xj1 = xj_ref[1]
    xj2 = xj_ref[2]
    out_ref[0] = (s1 * xj2 - s2 * xj1) * r
    out_ref[1] = (s2 * xj0 - s0 * xj2) * r
    out_ref[2] = (s0 * xj1 - s1 * xj0) * r


def _pick_be(e):
    for b in (640, 1280, 1920, 2560, 1600, 2000, 1000, 800, 500, 400, 320,
              256, 250, 200, 160, 128, 125, 100, 80, 64, 50, 40, 32, 25, 20,
              16, 10, 8, 5, 4, 2, 1):
        if e % b == 0:
            return b
    return 1


def kernel(x_j, edge_index, pos, W1, b1, W2, b2):
    e = edge_index.shape[1]
    n_nodes = pos.shape[0]
    c = x_j.shape[2]
    hidden = W1.shape[1]

    info = plsc.get_sparse_core_info()
    nc, ns = info.num_cores, info.num_subcores
    nw = nc * ns
    assert e % 128 == 0, "edge count must be a multiple of 128"
    b_per_w = ((e + nw * 128 - 1) // (nw * 128)) * 128

    i_arr = edge_index[0]
    j_arr = edge_index[1]

    vec8 = _make_sc_gather(n_nodes, e, b_per_w, nc)(
        pos.reshape(-1), i_arr, j_arr)

    xjt = jnp.transpose(x_j, (1, 0, 2))

    be = _pick_be(e)
    grid = (e // be,)
    out3 = pl.pallas_call(
        _tc_body,
        grid=grid,
        in_specs=[
            pl.BlockSpec((8, be), lambda i: (0, i)),
            pl.BlockSpec((3, be, c), lambda i: (0, i, 0)),
            pl.BlockSpec((1, hidden), lambda i: (0, 0)),
            pl.BlockSpec((1, hidden), lambda i: (0, 0)),
            pl.BlockSpec((hidden, c), lambda i: (0, 0)),
            pl.BlockSpec((1, c), lambda i: (0, 0)),
        ],
        out_specs=pl.BlockSpec((3, be, c), lambda i: (0, i, 0)),
        out_shape=jax.ShapeDtypeStruct((3, e, c), jnp.float32),
        compiler_params=pltpu.CompilerParams(
            dimension_semantics=("arbitrary",),
        ),
    )(vec8, xjt, W1, b1.reshape(1, hidden), W2, b2.reshape(1, c))
    return jnp.transpose(out3, (1, 0, 2))


# speedup vs baseline: 1.4216x; 1.4216x over previous
import functools
import math

import jax
import jax.numpy as jnp
from jax import lax
from jax.experimental import pallas as pl
from jax.experimental.pallas import tpu as pltpu
from jax.experimental.pallas import tpu_sc as plsc

_LANES = 16


def _make_sc_gather(n_nodes, e, b_per_w, nc):
    groups = b_per_w // _LANES

    mesh = plsc.VectorSubcoreMesh(core_axis_name="c", subcore_axis_name="s")

    @functools.partial(
        pl.kernel,
        out_type=jax.ShapeDtypeStruct((8, e), jnp.float32),
        mesh=mesh,
        scratch_types=[
            pltpu.VMEM((n_nodes * 3,), jnp.float32),
            pltpu.VMEM((b_per_w,), jnp.int32),
            pltpu.VMEM((b_per_w,), jnp.int32),
            pltpu.VMEM((8, b_per_w), jnp.float32),
        ],
        compiler_params=pltpu.CompilerParams(needs_layout_passes=False),
    )
    def sc_gather(pos_hbm, i_hbm, j_hbm, out_hbm, pos_v, i_v, j_v, vec_v):
        wid = lax.axis_index("s") * nc + lax.axis_index("c")
        base = jnp.minimum(wid * b_per_w, e - b_per_w)

        pltpu.sync_copy(pos_hbm, pos_v)
        pltpu.sync_copy(i_hbm.at[pl.ds(base, b_per_w)], i_v)
        pltpu.sync_copy(j_hbm.at[pl.ds(base, b_per_w)], j_v)

        @plsc.parallel_loop(0, groups, unroll=4)
        def body(g):
            off = g * _LANES
            i_idx = i_v[pl.ds(off, _LANES)] * 3
            j_idx = j_v[pl.ds(off, _LANES)] * 3
            comps = []
            for k in range(3):
                ip = plsc.load_gather(pos_v, [i_idx + k])
                jp = plsc.load_gather(pos_v, [j_idx + k])
                comps.append(ip - jp)
            v0, v1, v2 = comps
            d2 = v0 * v0 + v1 * v1 + v2 * v2
            for k, val in enumerate((v0, v1, v2, d2)):
                vec_v[k, pl.ds(off, _LANES)] = val

        pltpu.sync_copy(vec_v, out_hbm.at[:, pl.ds(base, b_per_w)])

    return sc_gather


_C_SH = math.sqrt(3.0 / (4.0 * math.pi))
_INV_SQRT2 = 1.0 / math.sqrt(2.0)


def _tc_body(vec_ref, xj_ref, w1_ref, b1_ref, w2_ref, b2_ref, out_ref):
    v = jnp.transpose(vec_ref[...])
    vx = v[:, 0:1]
    vy = v[:, 1:2]
    vz = v[:, 2:3]
    d2 = v[:, 3:4]
    dist = jnp.sqrt(d2 + 1e-12)
    inv = (_C_SH * _INV_SQRT2) / dist
    s0 = vy * inv
    s1 = vz * inv
    s2 = vx * inv
    a = dist * w1_ref[...] + b1_ref[...]
    h = a * (1.0 / (1.0 + jnp.exp(-a)))
    r = jnp.dot(h, w2_ref[...], preferred_element_type=jnp.float32) + b2_ref[...]
    xj0 = xj_ref[0]
    xj1 = xj_ref[1]
    xj2 = xj_ref[2]
    out_ref[0] = (s1 * xj2 - s2 * xj1) * r
    out_ref[1] = (s2 * xj0 - s0 * xj2) * r
    out_ref[2] = (s0 * xj1 - s1 * xj0) * r


def _pick_be(e):
    for b in (3200, 1280, 1920, 2560, 1600, 2000, 1000, 800, 500, 400, 320,
              256, 250, 200, 160, 128, 125, 100, 80, 64, 50, 40, 32, 25, 20,
              16, 10, 8, 5, 4, 2, 1):
        if e % b == 0:
            return b
    return 1


def kernel(x_j, edge_index, pos, W1, b1, W2, b2):
    e = edge_index.shape[1]
    n_nodes = pos.shape[0]
    c = x_j.shape[2]
    hidden = W1.shape[1]

    info = plsc.get_sparse_core_info()
    nc, ns = info.num_cores, info.num_subcores
    nw = nc * ns
    assert e % 128 == 0, "edge count must be a multiple of 128"
    b_per_w = ((e + nw * 128 - 1) // (nw * 128)) * 128

    i_arr = edge_index[0]
    j_arr = edge_index[1]

    vec8 = _make_sc_gather(n_nodes, e, b_per_w, nc)(
        pos.reshape(-1), i_arr, j_arr)

    xjt = jnp.transpose(x_j, (1, 0, 2))

    be = _pick_be(e)
    grid = (e // be,)
    out3 = pl.pallas_call(
        _tc_body,
        grid=grid,
        in_specs=[
            pl.BlockSpec((8, be), lambda i: (0, i)),
            pl.BlockSpec((3, be, c), lambda i: (0, i, 0)),
            pl.BlockSpec((1, hidden), lambda i: (0, 0)),
            pl.BlockSpec((1, hidden), lambda i: (0, 0)),
            pl.BlockSpec((hidden, c), lambda i: (0, 0)),
            pl.BlockSpec((1, c), lambda i: (0, 0)),
        ],
        out_specs=pl.BlockSpec((3, be, c), lambda i: (0, i, 0)),
        out_shape=jax.ShapeDtypeStruct((3, e, c), jnp.float32),
        compiler_params=pltpu.CompilerParams(
            dimension_semantics=("arbitrary",),
        ),
    )(vec8, xjt, W1, b1.reshape(1, hidden), W2, b2.reshape(1, c))
    return jnp.transpose(out3, (1, 0, 2))


# speedup vs baseline: 1.4522x; 1.0215x over previous
import functools
import math

import jax
import jax.numpy as jnp
from jax import lax
from jax.experimental import pallas as pl
from jax.experimental.pallas import tpu as pltpu
from jax.experimental.pallas import tpu_sc as plsc

_LANES = 16


def _make_sc_gather(n_nodes, e, b_per_w, nc):
    groups = b_per_w // _LANES

    mesh = plsc.VectorSubcoreMesh(core_axis_name="c", subcore_axis_name="s")

    @functools.partial(
        pl.kernel,
        out_type=jax.ShapeDtypeStruct((8, e), jnp.float32),
        mesh=mesh,
        scratch_types=[
            pltpu.VMEM((n_nodes * 3,), jnp.float32),
            pltpu.VMEM((b_per_w,), jnp.int32),
            pltpu.VMEM((b_per_w,), jnp.int32),
            pltpu.VMEM((8, b_per_w), jnp.float32),
        ],
        compiler_params=pltpu.CompilerParams(needs_layout_passes=False),
    )
    def sc_gather(pos_hbm, i_hbm, j_hbm, out_hbm, pos_v, i_v, j_v, vec_v):
        wid = lax.axis_index("s") * nc + lax.axis_index("c")
        base = jnp.minimum(wid * b_per_w, e - b_per_w)

        pltpu.sync_copy(pos_hbm, pos_v)
        pltpu.sync_copy(i_hbm.at[pl.ds(base, b_per_w)], i_v)
        pltpu.sync_copy(j_hbm.at[pl.ds(base, b_per_w)], j_v)

        @plsc.parallel_loop(0, groups, unroll=4)
        def body(g):
            off = g * _LANES
            i_idx = i_v[pl.ds(off, _LANES)] * 3
            j_idx = j_v[pl.ds(off, _LANES)] * 3
            comps = []
            for k in range(3):
                ip = plsc.load_gather(pos_v, [i_idx + k])
                jp = plsc.load_gather(pos_v, [j_idx + k])
                comps.append(ip - jp)
            v0, v1, v2 = comps
            d2 = v0 * v0 + v1 * v1 + v2 * v2
            for k, val in enumerate((v0, v1, v2, d2)):
                vec_v[k, pl.ds(off, _LANES)] = val

        pltpu.sync_copy(vec_v, out_hbm.at[:, pl.ds(base, b_per_w)])

    return sc_gather


_C_SH = math.sqrt(3.0 / (4.0 * math.pi))
_INV_SQRT2 = 1.0 / math.sqrt(2.0)


def _tc_body(vec_ref, xj_ref, w1_ref, b1_ref, w2_ref, b2_ref, out_ref):
    v = jnp.transpose(vec_ref[...])
    vx = v[:, 0:1]
    vy = v[:, 1:2]
    vz = v[:, 2:3]
    d2 = v[:, 3:4]
    dist = jnp.sqrt(d2 + 1e-12)
    inv = (_C_SH * _INV_SQRT2) / dist
    s0 = vy * inv
    s1 = vz * inv
    s2 = vx * inv
    a = dist * w1_ref[...] + b1_ref[...]
    h = a * (1.0 / (1.0 + jnp.exp(-a)))
    r = jnp.dot(h, w2_ref[...], preferred_element_type=jnp.float32) + b2_ref[...]
    xj0 = xj_ref[0]
    xj1 = xj_ref[1]
    xj2 = xj_ref[2]
    out_ref[0] = (s1 * xj2 - s2 * xj1) * r
    out_ref[1] = (s2 * xj0 - s0 * xj2) * r
    out_ref[2] = (s0 * xj1 - s1 * xj0) * r


def _pick_be(e):
    for b in (6400, 3200, 1280, 1920, 2560, 1600, 2000, 1000, 800, 500, 400, 320,
              256, 250, 200, 160, 128, 125, 100, 80, 64, 50, 40, 32, 25, 20,
              16, 10, 8, 5, 4, 2, 1):
        if e % b == 0:
            return b
    return 1


def kernel(x_j, edge_index, pos, W1, b1, W2, b2):
    e = edge_index.shape[1]
    n_nodes = pos.shape[0]
    c = x_j.shape[2]
    hidden = W1.shape[1]

    info = plsc.get_sparse_core_info()
    nc, ns = info.num_cores, info.num_subcores
    nw = nc * ns
    assert e % 128 == 0, "edge count must be a multiple of 128"
    b_per_w = ((e + nw * 128 - 1) // (nw * 128)) * 128

    i_arr = edge_index[0]
    j_arr = edge_index[1]

    vec8 = _make_sc_gather(n_nodes, e, b_per_w, nc)(
        pos.reshape(-1), i_arr, j_arr)

    xjt = jnp.transpose(x_j, (1, 0, 2))

    be = _pick_be(e)
    grid = (e // be,)
    out3 = pl.pallas_call(
        _tc_body,
        grid=grid,
        in_specs=[
            pl.BlockSpec((8, be), lambda i: (0, i)),
            pl.BlockSpec((3, be, c), lambda i: (0, i, 0)),
            pl.BlockSpec((1, hidden), lambda i: (0, 0)),
            pl.BlockSpec((1, hidden), lambda i: (0, 0)),
            pl.BlockSpec((hidden, c), lambda i: (0, 0)),
            pl.BlockSpec((1, c), lambda i: (0, 0)),
        ],
        out_specs=pl.BlockSpec((3, be, c), lambda i: (0, i, 0)),
        out_shape=jax.ShapeDtypeStruct((3, e, c), jnp.float32),
        compiler_params=pltpu.CompilerParams(
            dimension_semantics=("arbitrary",),
        ),
    )(vec8, xjt, W1, b1.reshape(1, hidden), W2, b2.reshape(1, c))
    return jnp.transpose(out3, (1, 0, 2))


# speedup vs baseline: 1.7671x; 1.2169x over previous
import functools
import math

import jax
import jax.numpy as jnp
from jax import lax
from jax.experimental import pallas as pl
from jax.experimental.pallas import tpu as pltpu
from jax.experimental.pallas import tpu_sc as plsc

_LANES = 16


def _make_sc_gather(n_nodes, e, b_per_w, nc):
    groups = b_per_w // _LANES

    mesh = plsc.VectorSubcoreMesh(core_axis_name="c", subcore_axis_name="s")

    @functools.partial(
        pl.kernel,
        out_type=jax.ShapeDtypeStruct((8, e), jnp.float32),
        mesh=mesh,
        scratch_types=[
            pltpu.VMEM((n_nodes * 3,), jnp.float32),
            pltpu.VMEM((b_per_w,), jnp.int32),
            pltpu.VMEM((b_per_w,), jnp.int32),
            pltpu.VMEM((8, b_per_w), jnp.float32),
        ],
        compiler_params=pltpu.CompilerParams(needs_layout_passes=False),
    )
    def sc_gather(pos_hbm, i_hbm, j_hbm, out_hbm, pos_v, i_v, j_v, vec_v):
        wid = lax.axis_index("s") * nc + lax.axis_index("c")
        base = jnp.minimum(wid * b_per_w, e - b_per_w)

        pltpu.sync_copy(pos_hbm, pos_v)
        pltpu.sync_copy(i_hbm.at[pl.ds(base, b_per_w)], i_v)
        pltpu.sync_copy(j_hbm.at[pl.ds(base, b_per_w)], j_v)

        @plsc.parallel_loop(0, groups, unroll=4)
        def body(g):
            off = g * _LANES
            i_idx = i_v[pl.ds(off, _LANES)] * 3
            j_idx = j_v[pl.ds(off, _LANES)] * 3
            comps = []
            for k in range(3):
                ip = plsc.load_gather(pos_v, [i_idx + k])
                jp = plsc.load_gather(pos_v, [j_idx + k])
                comps.append(ip - jp)
            v0, v1, v2 = comps
            d2 = v0 * v0 + v1 * v1 + v2 * v2
            for k, val in enumerate((v0, v1, v2, d2)):
                vec_v[k, pl.ds(off, _LANES)] = val

        pltpu.sync_copy(vec_v, out_hbm.at[:, pl.ds(base, b_per_w)])

    return sc_gather


_C_SH = math.sqrt(3.0 / (4.0 * math.pi))
_INV_SQRT2 = 1.0 / math.sqrt(2.0)


def _tc_body(vec_ref, xj_ref, w1_ref, b1_ref, w2_ref, b2_ref, out_ref):
    v8 = vec_ref[...]
    t = v8[3:4, :] + 1e-12
    ir = lax.rsqrt(t)
    inv = (_C_SH * _INV_SQRT2) * ir
    pack = jnp.concatenate(
        [v8[1:2, :] * inv, v8[2:3, :] * inv, v8[0:1, :] * inv, t * ir,
         v8[4:8, :]],
        axis=0,
    )
    p = jnp.transpose(pack)
    s0 = p[:, 0:1]
    s1 = p[:, 1:2]
    s2 = p[:, 2:3]
    dist = p[:, 3:4]
    a = dist * w1_ref[...] + b1_ref[...]
    h = a * (1.0 / (1.0 + jnp.exp(-a)))
    r = jnp.dot(h, w2_ref[...], preferred_element_type=jnp.float32) + b2_ref[...]
    xj0 = xj_ref[0]
    xj1 = xj_ref[1]
    xj2 = xj_ref[2]
    out_ref[0] = (s1 * xj2 - s2 * xj1) * r
    out_ref[1] = (s2 * xj0 - s0 * xj2) * r
    out_ref[2] = (s0 * xj1 - s1 * xj0) * r


def _pick_be(e):
    for b in (6400, 3200, 1280, 1920, 2560, 1600, 2000, 1000, 800, 500, 400, 320,
              256, 250, 200, 160, 128, 125, 100, 80, 64, 50, 40, 32, 25, 20,
              16, 10, 8, 5, 4, 2, 1):
        if e % b == 0:
            return b
    return 1


def kernel(x_j, edge_index, pos, W1, b1, W2, b2):
    e = edge_index.shape[1]
    n_nodes = pos.shape[0]
    c = x_j.shape[2]
    hidden = W1.shape[1]

    info = plsc.get_sparse_core_info()
    nc, ns = info.num_cores, info.num_subcores
    nw = nc * ns
    assert e % 128 == 0, "edge count must be a multiple of 128"
    b_per_w = ((e + nw * 128 - 1) // (nw * 128)) * 128

    i_arr = edge_index[0]
    j_arr = edge_index[1]

    vec8 = _make_sc_gather(n_nodes, e, b_per_w, nc)(
        pos.reshape(-1), i_arr, j_arr)

    xjt = jnp.transpose(x_j, (1, 0, 2))

    be = _pick_be(e)
    grid = (e // be,)
    out3 = pl.pallas_call(
        _tc_body,
        grid=grid,
        in_specs=[
            pl.BlockSpec((8, be), lambda i: (0, i)),
            pl.BlockSpec((3, be, c), lambda i: (0, i, 0)),
            pl.BlockSpec((1, hidden), lambda i: (0, 0)),
            pl.BlockSpec((1, hidden), lambda i: (0, 0)),
            pl.BlockSpec((hidden, c), lambda i: (0, 0)),
            pl.BlockSpec((1, c), lambda i: (0, 0)),
        ],
        out_specs=pl.BlockSpec((3, be, c), lambda i: (0, i, 0)),
        out_shape=jax.ShapeDtypeStruct((3, e, c), jnp.float32),
        compiler_params=pltpu.CompilerParams(
            dimension_semantics=("arbitrary",),
        ),
    )(vec8, xjt, W1, b1.reshape(1, hidden), W2, b2.reshape(1, c))
    return jnp.transpose(out3, (1, 0, 2))


# speedup vs baseline: 1.7681x; 1.0006x over previous
import functools
import math

import jax
import jax.numpy as jnp
from jax import lax
from jax.experimental import pallas as pl
from jax.experimental.pallas import tpu as pltpu
from jax.experimental.pallas import tpu_sc as plsc

_LANES = 16


def _make_sc_gather(n_nodes, e, b_per_w, nc):
    groups = b_per_w // _LANES

    mesh = plsc.VectorSubcoreMesh(core_axis_name="c", subcore_axis_name="s")

    @functools.partial(
        pl.kernel,
        out_type=jax.ShapeDtypeStruct((8, e), jnp.float32),
        mesh=mesh,
        scratch_types=[
            pltpu.VMEM((n_nodes * 3,), jnp.float32),
            pltpu.VMEM((b_per_w,), jnp.int32),
            pltpu.VMEM((b_per_w,), jnp.int32),
            pltpu.VMEM((8, b_per_w), jnp.float32),
        ],
        compiler_params=pltpu.CompilerParams(needs_layout_passes=False),
    )
    def sc_gather(pos_hbm, i_hbm, j_hbm, out_hbm, pos_v, i_v, j_v, vec_v):
        wid = lax.axis_index("s") * nc + lax.axis_index("c")
        base = jnp.minimum(wid * b_per_w, e - b_per_w)

        pltpu.sync_copy(pos_hbm, pos_v)
        pltpu.sync_copy(i_hbm.at[pl.ds(base, b_per_w)], i_v)
        pltpu.sync_copy(j_hbm.at[pl.ds(base, b_per_w)], j_v)

        @plsc.parallel_loop(0, groups, unroll=8)
        def body(g):
            off = g * _LANES
            i_idx = i_v[pl.ds(off, _LANES)] * 3
            j_idx = j_v[pl.ds(off, _LANES)] * 3
            comps = []
            for k in range(3):
                ip = plsc.load_gather(pos_v, [i_idx + k])
                jp = plsc.load_gather(pos_v, [j_idx + k])
                comps.append(ip - jp)
            v0, v1, v2 = comps
            d2 = v0 * v0 + v1 * v1 + v2 * v2
            for k, val in enumerate((v0, v1, v2, d2)):
                vec_v[k, pl.ds(off, _LANES)] = val

        pltpu.sync_copy(vec_v, out_hbm.at[:, pl.ds(base, b_per_w)])

    return sc_gather


_C_SH = math.sqrt(3.0 / (4.0 * math.pi))
_INV_SQRT2 = 1.0 / math.sqrt(2.0)


def _tc_body(vec_ref, xj_ref, w1_ref, b1_ref, w2_ref, b2_ref, out_ref):
    v8 = vec_ref[...]
    t = v8[3:4, :] + 1e-12
    ir = lax.rsqrt(t)
    inv = (_C_SH * _INV_SQRT2) * ir
    pack = jnp.concatenate(
        [v8[1:2, :] * inv, v8[2:3, :] * inv, v8[0:1, :] * inv, t * ir,
         v8[4:8, :]],
        axis=0,
    )
    p = jnp.transpose(pack)
    s0 = p[:, 0:1]
    s1 = p[:, 1:2]
    s2 = p[:, 2:3]
    dist = p[:, 3:4]
    a = dist * w1_ref[...] + b1_ref[...]
    h = a * (1.0 / (1.0 + jnp.exp(-a)))
    r = jnp.dot(h, w2_ref[...], preferred_element_type=jnp.float32) + b2_ref[...]
    xj0 = xj_ref[0]
    xj1 = xj_ref[1]
    xj2 = xj_ref[2]
    out_ref[0] = (s1 * xj2 - s2 * xj1) * r
    out_ref[1] = (s2 * xj0 - s0 * xj2) * r
    out_ref[2] = (s0 * xj1 - s1 * xj0) * r


def _pick_be(e):
    for b in (6400, 3200, 1280, 1920, 2560, 1600, 2000, 1000, 800, 500, 400, 320,
              256, 250, 200, 160, 128, 125, 100, 80, 64, 50, 40, 32, 25, 20,
              16, 10, 8, 5, 4, 2, 1):
        if e % b == 0:
            return b
    return 1


def kernel(x_j, edge_index, pos, W1, b1, W2, b2):
    e = edge_index.shape[1]
    n_nodes = pos.shape[0]
    c = x_j.shape[2]
    hidden = W1.shape[1]

    info = plsc.get_sparse_core_info()
    nc, ns = info.num_cores, info.num_subcores
    nw = nc * ns
    assert e % 128 == 0, "edge count must be a multiple of 128"
    b_per_w = ((e + nw * 128 - 1) // (nw * 128)) * 128

    i_arr = edge_index[0]
    j_arr = edge_index[1]

    vec8 = _make_sc_gather(n_nodes, e, b_per_w, nc)(
        pos.reshape(-1), i_arr, j_arr)

    xjt = jnp.transpose(x_j, (1, 0, 2))

    be = _pick_be(e)
    grid = (e // be,)
    out3 = pl.pallas_call(
        _tc_body,
        grid=grid,
        in_specs=[
            pl.BlockSpec((8, be), lambda i: (0, i)),
            pl.BlockSpec((3, be, c), lambda i: (0, i, 0)),
            pl.BlockSpec((1, hidden), lambda i: (0, 0)),
            pl.BlockSpec((1, hidden), lambda i: (0, 0)),
            pl.BlockSpec((hidden, c), lambda i: (0, 0)),
            pl.BlockSpec((1, c), lambda i: (0, 0)),
        ],
        out_specs=pl.BlockSpec((3, be, c), lambda i: (0, i, 0)),
        out_shape=jax.ShapeDtypeStruct((3, e, c), jnp.float32),
        compiler_params=pltpu.CompilerParams(
            dimension_semantics=("parallel",),
        ),
    )(vec8, xjt, W1, b1.reshape(1, hidden), W2, b2.reshape(1, c))
    return jnp.transpose(out3, (1, 0, 2))
